# baseline (speedup 1.0000x reference)
import functools

import jax
import jax.numpy as jnp
from jax.experimental import pallas as pl
from jax.experimental.pallas import tpu as pltpu

_TOP_K = 8


def _gate_topk_body(x_ref, w_ref, b_ref, w_out_ref, i_out_ref):
    g = (
        jax.lax.dot_general(
            w_ref[...],
            x_ref[...],
            dimension_numbers=(((1,), (1,)), ((), ())),
            preferred_element_type=jnp.float32,
        )
        + b_ref[...]
    )
    sub = jax.lax.broadcasted_iota(jnp.int32, g.shape, 0)
    cur = g
    vals = []
    idxs = []
    for k in range(_TOP_K):
        v, i = cur, sub
        while v.shape[0] > 1:
            h = v.shape[0] // 2
            c = v[:h] >= v[h:]
            v = jnp.where(c, v[:h], v[h:])
            i = jnp.where(c, i[:h], i[h:])
        vals.append(v)
        idxs.append(i)
        if k + 1 < _TOP_K:
            cur = jnp.where(sub == i, -jnp.inf, cur)
    v = jnp.concatenate(vals, axis=0)
    ew = jnp.exp(v - v[0:1, :])
    w_out_ref[...] = ew / jnp.sum(ew, axis=0, keepdims=True)
    i_out_ref[...] = jnp.concatenate(idxs, axis=0)


@functools.partial(jax.jit, static_argnames=("block_t", "interpret"))
def _gate_topk(x, W, b, block_t=2048, interpret=False):
    n_tokens, hidden = x.shape
    n_experts = W.shape[0]
    b2 = b.reshape(n_experts, 1)
    grid = (n_tokens // block_t,)
    w_out, i_out = pl.pallas_call(
        _gate_topk_body,
        grid=grid,
        in_specs=[
            pl.BlockSpec((block_t, hidden), lambda i: (i, 0)),
            pl.BlockSpec((n_experts, hidden), lambda i: (0, 0)),
            pl.BlockSpec((n_experts, 1), lambda i: (0, 0)),
        ],
        out_specs=[
            pl.BlockSpec((_TOP_K, block_t), lambda i: (0, i)),
            pl.BlockSpec((_TOP_K, block_t), lambda i: (0, i)),
        ],
        out_shape=[
            jax.ShapeDtypeStruct((_TOP_K, n_tokens), jnp.float32),
            jax.ShapeDtypeStruct((_TOP_K, n_tokens), jnp.int32),
        ],
        interpret=interpret,
    )(x, W, b2)
    return w_out.T, i_out.T


def kernel(x, W, b):
    return _gate_topk(x, W, b)


# speedup vs baseline: 1.0600x; 1.0600x over previous
import functools

import jax
import jax.numpy as jnp
from jax.experimental import pallas as pl
from jax.experimental.pallas import tpu as pltpu

_TOP_K = 8


def _gate_topk_body(x_ref, w_ref, b_ref, w_out_ref, i_out_ref):
    g = (
        jax.lax.dot_general(
            w_ref[...],
            x_ref[...],
            dimension_numbers=(((1,), (1,)), ((), ())),
            preferred_element_type=jnp.float32,
        )
        + b_ref[...]
    )
    sub = jax.lax.broadcasted_iota(jnp.int32, g.shape, 0)
    cur = g
    vals = []
    idxs = []
    for k in range(_TOP_K):
        v, i = cur, sub
        while v.shape[0] > 1:
            h = v.shape[0] // 2
            c = v[:h] >= v[h:]
            v = jnp.where(c, v[:h], v[h:])
            i = jnp.where(c, i[:h], i[h:])
        vals.append(v)
        idxs.append(i)
        if k + 1 < _TOP_K:
            cur = jnp.where(sub == i, -jnp.inf, cur)
    v = jnp.concatenate(vals, axis=0)
    ew = jnp.exp(v - v[0:1, :])
    w_out_ref[...] = ew / jnp.sum(ew, axis=0, keepdims=True)
    i_out_ref[...] = jnp.concatenate(idxs, axis=0)


@functools.partial(jax.jit, static_argnames=("block_t", "interpret"))
def _gate_topk(x, W, b, block_t=8192, interpret=False):
    n_tokens, hidden = x.shape
    n_experts = W.shape[0]
    b2 = b.reshape(n_experts, 1)
    grid = (n_tokens // block_t,)
    w_out, i_out = pl.pallas_call(
        _gate_topk_body,
        grid=grid,
        in_specs=[
            pl.BlockSpec((block_t, hidden), lambda i: (i, 0)),
            pl.BlockSpec((n_experts, hidden), lambda i: (0, 0)),
            pl.BlockSpec((n_experts, 1), lambda i: (0, 0)),
        ],
        out_specs=[
            pl.BlockSpec((_TOP_K, block_t), lambda i: (0, i)),
            pl.BlockSpec((_TOP_K, block_t), lambda i: (0, i)),
        ],
        out_shape=[
            jax.ShapeDtypeStruct((_TOP_K, n_tokens), jnp.float32),
            jax.ShapeDtypeStruct((_TOP_K, n_tokens), jnp.int32),
        ],
        interpret=interpret,
    )(x, W, b2)
    return w_out.T, i_out.T


def kernel(x, W, b):
    return _gate_topk(x, W, b)


# speedup vs baseline: 1.0916x; 1.0298x over previous
import functools

import jax
import jax.numpy as jnp
from jax.experimental import pallas as pl
from jax.experimental.pallas import tpu as pltpu

_TOP_K = 8


def _gate_topk_body(x_ref, w_ref, b_ref, w_out_ref, i_out_ref):
    g = (
        jax.lax.dot_general(
            w_ref[...],
            x_ref[...],
            dimension_numbers=(((1,), (1,)), ((), ())),
            preferred_element_type=jnp.float32,
        )
        + b_ref[...]
    )
    sub = jax.lax.broadcasted_iota(jnp.int32, g.shape, 0)
    cur = g
    vals = []
    idxs = []
    for k in range(_TOP_K):
        v, i = cur, sub
        while v.shape[0] > 1:
            h = v.shape[0] // 2
            c = v[:h] >= v[h:]
            v = jnp.where(c, v[:h], v[h:])
            i = jnp.where(c, i[:h], i[h:])
        vals.append(v)
        idxs.append(i)
        if k + 1 < _TOP_K:
            cur = jnp.where(sub == i, -jnp.inf, cur)
    v = jnp.concatenate(vals, axis=0)
    ew = jnp.exp(v - v[0:1, :])
    w_out_ref[...] = ew / jnp.sum(ew, axis=0, keepdims=True)
    i_out_ref[...] = jnp.concatenate(idxs, axis=0)


@functools.partial(jax.jit, static_argnames=("block_t", "interpret"))
def _gate_topk(x, W, b, block_t=4096, interpret=False):
    n_tokens, hidden = x.shape
    n_experts = W.shape[0]
    b2 = b.reshape(n_experts, 1)
    grid = (n_tokens // block_t,)
    w_out, i_out = pl.pallas_call(
        _gate_topk_body,
        grid=grid,
        in_specs=[
            pl.BlockSpec((block_t, hidden), lambda i: (i, 0)),
            pl.BlockSpec((n_experts, hidden), lambda i: (0, 0)),
            pl.BlockSpec((n_experts, 1), lambda i: (0, 0)),
        ],
        out_specs=[
            pl.BlockSpec((_TOP_K, block_t), lambda i: (0, i)),
            pl.BlockSpec((_TOP_K, block_t), lambda i: (0, i)),
        ],
        out_shape=[
            jax.ShapeDtypeStruct((_TOP_K, n_tokens), jnp.float32),
            jax.ShapeDtypeStruct((_TOP_K, n_tokens), jnp.int32),
        ],
        interpret=interpret,
    )(x, W, b2)
    return w_out.T, i_out.T


def kernel(x, W, b):
    return _gate_topk(x, W, b)


# speedup vs baseline: 1.0936x; 1.0018x over previous
import functools

import jax
import jax.numpy as jnp
from jax.experimental import pallas as pl

_TOP_K = 8


def _gate_topk_body(x_ref, w_ref, b_ref, w_out_ref, i_out_ref):
    g = (
        jax.lax.dot_general(
            w_ref[...],
            x_ref[...],
            dimension_numbers=(((1,), (1,)), ((), ())),
            preferred_element_type=jnp.float32,
        )
        + b_ref[...]
    )
    sub = jax.lax.broadcasted_iota(jnp.int32, g.shape, 0)
    cur = g
    vals = []
    idxs = []
    for k in range(_TOP_K):
        v, i = cur, sub
        while v.shape[0] > 1:
            h = v.shape[0] // 2
            c = v[:h] >= v[h:]
            v = jnp.where(c, v[:h], v[h:])
            i = jnp.where(c, i[:h], i[h:])
        vals.append(v)
        idxs.append(i)
        if k + 1 < _TOP_K:
            cur = jnp.where(sub == i, -jnp.inf, cur)
    v = jnp.concatenate(vals, axis=0)
    ew = jnp.exp(v - v[0:1, :])
    w_out_ref[...] = ew / jnp.sum(ew, axis=0, keepdims=True)
    i_out_ref[...] = jnp.concatenate(idxs, axis=0)


@functools.partial(jax.jit, static_argnames=("block_t",))
def _gate_topk(x, W, b, block_t=4096):
    n_tokens, hidden = x.shape
    n_experts = W.shape[0]
    b2 = b.reshape(n_experts, 1)
    grid = (n_tokens // block_t,)
    w_out, i_out = pl.pallas_call(
        _gate_topk_body,
        grid=grid,
        in_specs=[
            pl.BlockSpec((block_t, hidden), lambda i: (i, 0)),
            pl.BlockSpec((n_experts, hidden), lambda i: (0, 0)),
            pl.BlockSpec((n_experts, 1), lambda i: (0, 0)),
        ],
        out_specs=[
            pl.BlockSpec((_TOP_K, block_t), lambda i: (0, i)),
            pl.BlockSpec((_TOP_K, block_t), lambda i: (0, i)),
        ],
        out_shape=[
            jax.ShapeDtypeStruct((_TOP_K, n_tokens), jnp.float32),
            jax.ShapeDtypeStruct((_TOP_K, n_tokens), jnp.int32),
        ],
    )(x, W, b2)
    return w_out.T, i_out.T


def kernel(x, W, b):
    return _gate_topk(x, W, b)
